# Initial kernel scaffold; baseline (speedup 1.0000x reference)
import jax
import jax.numpy as jnp
from jax.experimental import pallas as pl


def kernel(input, table):
    raise NotImplementedError("write your pallas kernel here")


# speedup vs baseline: 1.4940x; 1.4940x over previous
import functools

import jax
import jax.numpy as jnp
from jax import lax
from jax.experimental import pallas as pl
from jax.experimental.pallas import tpu as pltpu
from jax.experimental.pallas import tpu_sc as plsc

EMBED_DIM = 32
NUM_CORES = 2
NUM_SUBCORES = 16
NUM_WORKERS = NUM_CORES * NUM_SUBCORES
CHUNK = 1280


@functools.partial(jax.jit, static_argnums=(2, 3))
def _gather_sc(idx, table, n_per_w, n_chunks):
    mesh = plsc.VectorSubcoreMesh(core_axis_name="c", subcore_axis_name="s")
    n_total = n_per_w * NUM_WORKERS

    @functools.partial(
        pl.kernel,
        mesh=mesh,
        out_type=jax.ShapeDtypeStruct((n_total, EMBED_DIM), jnp.float32),
        scratch_types=[
            pltpu.VMEM((n_chunks, CHUNK), jnp.int32),
            pltpu.VMEM((CHUNK, EMBED_DIM), jnp.float32),
            pltpu.VMEM((CHUNK, EMBED_DIM), jnp.float32),
            pltpu.SemaphoreType.DMA,
            pltpu.SemaphoreType.DMA,
            pltpu.SemaphoreType.DMA,
            pltpu.SemaphoreType.DMA,
        ],
        compiler_params=pltpu.CompilerParams(use_tc_tiling_on_sc=False),
    )
    def k(idx_hbm, table_hbm, out_hbm, idx_v, rows0, rows1, g0, g1, o0, o1):
        rows = (rows0, rows1)
        gsem = (g0, g1)
        osem = (o0, o1)
        wid = lax.axis_index("s") * NUM_CORES + lax.axis_index("c")
        base = wid * n_per_w

        pltpu.sync_copy(idx_hbm.at[wid], idx_v)

        def gather_start(c, b):
            pltpu.async_copy(table_hbm.at[idx_v.at[c]], rows[b], gsem[b])

        def gather_wait(b):
            pltpu.make_async_copy(
                table_hbm.at[pl.ds(0, CHUNK)], rows[b], gsem[b]
            ).wait()

        def store_start(c, b):
            pltpu.async_copy(
                rows[b], out_hbm.at[pl.ds(base + c * CHUNK, CHUNK)], osem[b]
            )

        def store_wait(b):
            pltpu.make_async_copy(
                rows[b], out_hbm.at[pl.ds(base, CHUNK)], osem[b]
            ).wait()

        gather_start(0, 0)
        gather_wait(0)
        gather_start(1, 1)
        store_start(0, 0)

        def body(g, carry):
            for j in range(2):
                c = 2 * g + 1 + j
                b = (1 + j) % 2
                gather_wait(b)
                store_wait(1 - b)
                gather_start(c + 1, 1 - b)
                store_start(c, b)
            return carry

        lax.fori_loop(0, (n_chunks - 2) // 2, body, 0)

        gather_wait(1)
        store_start(n_chunks - 1, 1)
        store_wait(0)
        store_wait(1)

    return k(idx, table)


def kernel(input, table):
    batch, hist = input.shape
    n_total = batch * hist
    n_per_w = n_total // NUM_WORKERS
    n_chunks = n_per_w // CHUNK
    idx = input.reshape(NUM_WORKERS, n_chunks, CHUNK).astype(jnp.int32)
    out = _gather_sc(idx, table, n_per_w, n_chunks)
    return out.reshape(batch, hist, EMBED_DIM)
